# baseline (speedup 1.0000x reference)
import functools

import jax
import jax.numpy as jnp
from jax import lax
from jax.experimental import pallas as pl
from jax.experimental.pallas import tpu as pltpu

DM = 2048
DS = 32768
TK = 32
CHUNK = 16

TOK_BLK = 128
DICT_BLK = 2048
DEC_KBLK = 2048


def _encode_body(x_ref, w_ref, b_ref, pre_ref, cmax_ref):
    acc = jax.lax.dot_general(
        x_ref[...], w_ref[...],
        dimension_numbers=(((1,), (1,)), ((), ())),
        preferred_element_type=jnp.float32,
    )
    acc = acc + b_ref[...][None, :]
    pre_ref[...] = acc
    cmax_ref[...] = jnp.max(
        acc.reshape(TOK_BLK, DICT_BLK // CHUNK, CHUNK), axis=2)


def _encode(x_flat, W_enc, b_enc):
    nt = DM * 1 and x_flat.shape[0]
    grid = (nt // TOK_BLK, DS // DICT_BLK)
    return pl.pallas_call(
        _encode_body,
        grid=grid,
        in_specs=[
            pl.BlockSpec((TOK_BLK, DM), lambda i, j: (i, 0)),
            pl.BlockSpec((DICT_BLK, DM), lambda i, j: (j, 0)),
            pl.BlockSpec((DICT_BLK,), lambda i, j: (j,)),
        ],
        out_specs=[
            pl.BlockSpec((TOK_BLK, DICT_BLK), lambda i, j: (i, j)),
            pl.BlockSpec((TOK_BLK, DICT_BLK // CHUNK), lambda i, j: (i, j)),
        ],
        out_shape=[
            jax.ShapeDtypeStruct((nt, DS), jnp.float32),
            jax.ShapeDtypeStruct((nt, DS // CHUNK), jnp.float32),
        ],
    )(x_flat, W_enc, b_enc)


def _decode_body(f_ref, w_ref, b_ref, out_ref):
    k = pl.program_id(1)
    acc = jax.lax.dot_general(
        f_ref[...], w_ref[...],
        dimension_numbers=(((1,), (1,)), ((), ())),
        preferred_element_type=jnp.float32,
    )

    @pl.when(k == 0)
    def _():
        out_ref[...] = acc + b_ref[...][None, :]

    @pl.when(k != 0)
    def _():
        out_ref[...] += acc


def _decode(features, W_dec, b_dec):
    nt = features.shape[0]
    grid = (nt // TOK_BLK, DS // DEC_KBLK)
    return pl.pallas_call(
        _decode_body,
        grid=grid,
        in_specs=[
            pl.BlockSpec((TOK_BLK, DEC_KBLK), lambda i, k: (i, k)),
            pl.BlockSpec((DM, DEC_KBLK), lambda i, k: (0, k)),
            pl.BlockSpec((DM,), lambda i, k: (0,)),
        ],
        out_specs=pl.BlockSpec((TOK_BLK, DM), lambda i, k: (i, 0)),
        out_shape=jax.ShapeDtypeStruct((nt, DM), jnp.float32),
        compiler_params=pltpu.CompilerParams(
            dimension_semantics=("parallel", "arbitrary"),
        ),
    )(features, W_dec, b_dec)


def _topk_features(pre_acts, cmax):
    vals, idx = jax.lax.top_k(pre_acts, TK)
    rows = jnp.arange(pre_acts.shape[0])[:, None]
    features = jnp.zeros_like(pre_acts).at[rows, idx].set(
        jnp.maximum(vals, 0.0))
    return features


def kernel(x, W_enc, b_enc, W_dec, b_dec):
    orig_shape = x.shape
    x_flat = x.reshape(-1, DM)
    pre_acts, cmax = _encode(x_flat, W_enc, b_enc)
    features = _topk_features(pre_acts, cmax)
    recon = _decode(features, W_dec, b_dec)
    return (
        recon.reshape(orig_shape),
        features.reshape(orig_shape[:-1] + (DS,)),
        pre_acts.reshape(orig_shape[:-1] + (DS,)),
    )


# speedup vs baseline: 2.5867x; 2.5867x over previous
import functools

import jax
import jax.numpy as jnp
from jax import lax
from jax.experimental import pallas as pl
from jax.experimental.pallas import tpu as pltpu

DM = 2048
DS = 32768
TK = 32
CHUNK = 16

TOK_BLK = 128
DICT_BLK = 2048
DEC_KBLK = 2048


def _encode_body(x_ref, w_ref, b_ref, pre_ref, cmax_ref):
    acc = jax.lax.dot_general(
        x_ref[...], w_ref[...],
        dimension_numbers=(((1,), (1,)), ((), ())),
        preferred_element_type=jnp.float32,
    )
    acc = acc + b_ref[...][None, :]
    pre_ref[...] = acc
    cmax_ref[...] = jnp.max(
        acc.reshape(TOK_BLK, DICT_BLK // CHUNK, CHUNK), axis=2)


def _encode(x_flat, W_enc, b_enc):
    nt = DM * 1 and x_flat.shape[0]
    grid = (nt // TOK_BLK, DS // DICT_BLK)
    return pl.pallas_call(
        _encode_body,
        grid=grid,
        in_specs=[
            pl.BlockSpec((TOK_BLK, DM), lambda i, j: (i, 0)),
            pl.BlockSpec((DICT_BLK, DM), lambda i, j: (j, 0)),
            pl.BlockSpec((DICT_BLK,), lambda i, j: (j,)),
        ],
        out_specs=[
            pl.BlockSpec((TOK_BLK, DICT_BLK), lambda i, j: (i, j)),
            pl.BlockSpec((TOK_BLK, DICT_BLK // CHUNK), lambda i, j: (i, j)),
        ],
        out_shape=[
            jax.ShapeDtypeStruct((nt, DS), jnp.float32),
            jax.ShapeDtypeStruct((nt, DS // CHUNK), jnp.float32),
        ],
    )(x_flat, W_enc, b_enc)


def _decode_body(f_ref, w_ref, b_ref, out_ref):
    k = pl.program_id(1)
    acc = jax.lax.dot_general(
        f_ref[...], w_ref[...],
        dimension_numbers=(((1,), (1,)), ((), ())),
        preferred_element_type=jnp.float32,
    )

    @pl.when(k == 0)
    def _():
        out_ref[...] = acc + b_ref[...][None, :]

    @pl.when(k != 0)
    def _():
        out_ref[...] += acc


def _decode(features, W_dec, b_dec):
    nt = features.shape[0]
    grid = (nt // TOK_BLK, DS // DEC_KBLK)
    return pl.pallas_call(
        _decode_body,
        grid=grid,
        in_specs=[
            pl.BlockSpec((TOK_BLK, DEC_KBLK), lambda i, k: (i, k)),
            pl.BlockSpec((DM, DEC_KBLK), lambda i, k: (0, k)),
            pl.BlockSpec((DM,), lambda i, k: (0,)),
        ],
        out_specs=pl.BlockSpec((TOK_BLK, DM), lambda i, k: (i, 0)),
        out_shape=jax.ShapeDtypeStruct((nt, DM), jnp.float32),
        compiler_params=pltpu.CompilerParams(
            dimension_semantics=("parallel", "arbitrary"),
        ),
    )(features, W_dec, b_dec)


NEG = -3.0e38
NW = 32
RPW = 2048 // NW
NCHV = (DS // CHUNK) // 16
POOL = 14


def _splat_lane(v, lane):
    idx = jnp.full((16,), lane, jnp.int32)
    return jnp.take(v, idx)


def _merge2(a, b, sort_loser):
    from jax.experimental.pallas import tpu_sc as plsc
    av, ai = a
    bv, bi = b
    rbv = lax.rev(bv, (0,))
    rbi = lax.rev(bi, (0,))
    c = av >= rbv
    hv = jnp.where(c, av, rbv)
    hi = jnp.where(c, ai, rbi)
    lv = jnp.where(c, rbv, av)
    li = jnp.where(c, rbi, ai)
    wv, wi = plsc.sort_key_val(hv, hi, descending=True)
    if sort_loser:
        lv, li = plsc.sort_key_val(lv, li, descending=True)
    return (wv, wi), (lv, li)


def _tree_top16(leaves, collect_losers):
    losers = []
    cur = list(leaves)
    while len(cur) > 1:
        nxt = []
        for i in range(0, len(cur) - 1, 2):
            w, l = _merge2(cur[i], cur[i + 1], sort_loser=collect_losers)
            nxt.append(w)
            if collect_losers:
                losers.append(l)
        if len(cur) % 2:
            nxt.append(cur[-1])
        cur = nxt
    return cur[0], losers


def _compact(pool_v, pool_i, Av, Ai, Bv, Bi):
    from jax.experimental.pallas import tpu_sc as plsc
    leaves = []
    for s in range(POOL):
        pv = pool_v[pl.ds(s * 16, 16)]
        pi = pool_i[pl.ds(s * 16, 16)]
        leaves.append((pv, pi))
    leaves.append((Av, Ai))
    leaves.append((Bv, Bi))
    (nAv, nAi), losers = _tree_top16(leaves, collect_losers=True)
    (nBv, nBi), _ = _tree_top16(losers, collect_losers=False)
    thr = _splat_lane(nBv, 15)
    return nAv, nAi, nBv, nBi, thr


def _scan_level(read_vreg, read_idx, nv, pool_v, pool_i, thr0, Av, Ai, Bv, Bi):
    slot0 = jnp.int32(0)

    @pl.loop(0, nv, init_carry=(slot0, thr0, Av, Ai, Bv, Bi))
    def scan(j, carry):
        slot, thr, av, ai, bv, bi = carry
        from jax.experimental.pallas import tpu_sc as plsc
        v = read_vreg(j)
        vs, is_ = plsc.sort_key_val(v, read_idx(j), descending=True)
        hit = vs[0] >= thr[0]

        def on_hit(slot, thr, av, ai, bv, bi):
            vm = jnp.where(vs >= thr, vs, NEG)
            pool_v[pl.ds(slot * 16, 16)] = vm
            pool_i[pl.ds(slot * 16, 16)] = is_
            slot = slot + 1

            def do_compact(av, ai, bv, bi):
                nav, nai, nbv, nbi, nthr = _compact(
                    pool_v, pool_i, av, ai, bv, bi)
                return jnp.int32(0), nthr, nav, nai, nbv, nbi

            def no_compact(av, ai, bv, bi):
                return slot, thr, av, ai, bv, bi

            return lax.cond(slot == POOL, do_compact, no_compact,
                            av, ai, bv, bi)

        def no_hit(slot, thr, av, ai, bv, bi):
            return slot, thr, av, ai, bv, bi

        return lax.cond(hit, on_hit, no_hit, slot, thr, av, ai, bv, bi)

    slot, thr, av, ai, bv, bi = scan
    neg16 = jnp.broadcast_to(NEG, (16,))
    zero16 = jnp.zeros((16,), jnp.int32)

    @pl.loop(slot, POOL)
    def _(s):
        pool_v[pl.ds(s * 16, 16)] = neg16
        pool_i[pl.ds(s * 16, 16)] = zero16

    def do_final(av, ai, bv, bi):
        nav, nai, nbv, nbi, nthr = _compact(pool_v, pool_i, av, ai, bv, bi)
        return nthr, nav, nai, nbv, nbi

    def no_final(av, ai, bv, bi):
        return thr, av, ai, bv, bi

    return lax.cond(slot > 0, do_final, no_final, av, ai, bv, bi)


def _sc_topk_body(cmax_hbm, prech_hbm, feat_hbm,
                  cm_buf, pool_v, pool_i, gidx, gbuf, eid, feat_buf,
                  last_ids, sem_g, sem_w0, sem_w1):
    from jax.experimental.pallas import tpu_sc as plsc
    wid = lax.axis_index("s") * 2 + lax.axis_index("c")
    iota = lax.iota(jnp.int32, 16)
    neg16 = jnp.broadcast_to(NEG, (16,))
    zero16i = jnp.zeros((16,), jnp.int32)
    zero16f = jnp.zeros((16,), jnp.float32)

    @pl.loop(0, 2 * DS // 16)
    def _(i):
        feat_buf[pl.ds(i * 16, 16)] = zero16f

    last_ids[pl.ds(0, 16)] = iota
    last_ids[pl.ds(16, 16)] = iota + 16
    last_ids[pl.ds(32, 16)] = iota
    last_ids[pl.ds(48, 16)] = iota + 16

    @pl.loop(0, RPW)
    def _(rr):
        row = wid * RPW + rr
        pltpu.sync_copy(cmax_hbm.at[row], cm_buf)

        thr, av, ai, bv, bi = _scan_level(
            lambda j: cm_buf[pl.ds(j * 16, 16)],
            lambda j: iota + j * 16,
            NCHV, pool_v, pool_i,
            neg16, neg16, zero16i, neg16, zero16i)

        base = row * (DS // CHUNK)
        gidx[pl.ds(0, 16)] = ai + base
        gidx[pl.ds(16, 16)] = bi + base
        pltpu.async_copy(prech_hbm.at[gidx], gbuf, sem_g).wait()

        for c in range(16):
            eid[pl.ds(c * 16, 16)] = _splat_lane(ai, c) * 16 + iota
            eid[pl.ds((c + 16) * 16, 16)] = _splat_lane(bi, c) * 16 + iota

        def read_v(c):
            return gbuf[c]

        def read_idx2(c):
            return eid[pl.ds(c * 16, 16)]

        thr2, av2, ai2, bv2, bi2 = _scan_level(
            read_v, read_idx2, 32, pool_v, pool_i,
            thr, neg16, zero16i, neg16, zero16i)

        b = rr % 2
        off = b * DS

        def wait_prev(_):
            prev_row = row - 2
            pltpu.make_async_copy(
                feat_buf.at[pl.ds(off, DS)], feat_hbm.at[prev_row],
                sem_w0).wait()
            return 0

        def wait_prev1(_):
            prev_row = row - 2
            pltpu.make_async_copy(
                feat_buf.at[pl.ds(off, DS)], feat_hbm.at[prev_row],
                sem_w1).wait()
            return 0

        def no_wait(_):
            return 0

        _ = lax.cond(rr >= 2,
                     lambda _: lax.cond(b == 0, wait_prev, wait_prev1, 0),
                     no_wait, 0)

        li0 = last_ids[pl.ds(b * 32, 16)]
        li1 = last_ids[pl.ds(b * 32 + 16, 16)]
        plsc.store_scatter(feat_buf, [li0 + off], zero16f)
        plsc.store_scatter(feat_buf, [li1 + off], zero16f)
        va = jnp.maximum(av2, 0.0)
        vb = jnp.maximum(bv2, 0.0)
        plsc.store_scatter(feat_buf, [ai2 + off], va)
        plsc.store_scatter(feat_buf, [bi2 + off], vb)
        last_ids[pl.ds(b * 32, 16)] = ai2
        last_ids[pl.ds(b * 32 + 16, 16)] = bi2

        def start0(_):
            pltpu.async_copy(feat_buf.at[pl.ds(off, DS)],
                             feat_hbm.at[row], sem_w0)
            return 0

        def start1(_):
            pltpu.async_copy(feat_buf.at[pl.ds(off, DS)],
                             feat_hbm.at[row], sem_w1)
            return 0

        _ = lax.cond(b == 0, start0, start1, 0)

    last0 = wid * RPW + RPW - 2
    pltpu.make_async_copy(feat_buf.at[pl.ds(0, DS)],
                          feat_hbm.at[last0], sem_w0).wait()
    pltpu.make_async_copy(feat_buf.at[pl.ds(DS, DS)],
                          feat_hbm.at[last0 + 1], sem_w1).wait()


def _sc_topk(cmax, pre_chunks):
    from jax.experimental.pallas import tpu_sc as plsc
    mesh = plsc.VectorSubcoreMesh(core_axis_name="c", subcore_axis_name="s",
                                  num_cores=2, num_subcores=16)
    nt = cmax.shape[0]
    return pl.kernel(
        _sc_topk_body,
        out_type=jax.ShapeDtypeStruct((nt, DS), jnp.float32),
        mesh=mesh,
        compiler_params=pltpu.CompilerParams(
            needs_layout_passes=False, use_tc_tiling_on_sc=False),
        scratch_types=[
            pltpu.VMEM((DS // CHUNK,), jnp.float32),
            pltpu.VMEM((POOL * 16,), jnp.float32),
            pltpu.VMEM((POOL * 16,), jnp.int32),
            pltpu.VMEM((32,), jnp.int32),
            pltpu.VMEM((32, 16), jnp.float32),
            pltpu.VMEM((32 * 16,), jnp.int32),
            pltpu.VMEM((2 * DS,), jnp.float32),
            pltpu.VMEM((64,), jnp.int32),
            pltpu.SemaphoreType.DMA,
            pltpu.SemaphoreType.DMA,
            pltpu.SemaphoreType.DMA,
        ],
    )(cmax, pre_chunks)


def _topk_features(pre_acts, cmax):
    pre_chunks = pre_acts.reshape(-1, CHUNK)
    return _sc_topk(cmax, pre_chunks)


def kernel(x, W_enc, b_enc, W_dec, b_dec):
    orig_shape = x.shape
    x_flat = x.reshape(-1, DM)
    pre_acts, cmax = _encode(x_flat, W_enc, b_enc)
    features = _topk_features(pre_acts, cmax)
    recon = _decode(features, W_dec, b_dec)
    return (
        recon.reshape(orig_shape),
        features.reshape(orig_shape[:-1] + (DS,)),
        pre_acts.reshape(orig_shape[:-1] + (DS,)),
    )


# speedup vs baseline: 3.2904x; 1.2721x over previous
import functools

import jax
import jax.numpy as jnp
from jax import lax
from jax.experimental import pallas as pl
from jax.experimental.pallas import tpu as pltpu

DM = 2048
DS = 32768
TK = 32
CHUNK = 16

TOK_BLK = 128
DICT_BLK = 2048
DEC_KBLK = 1024


def _encode_body(x_ref, w_ref, b_ref, pre_ref, cmax_ref):
    acc = jax.lax.dot_general(
        x_ref[...], w_ref[...],
        dimension_numbers=(((1,), (1,)), ((), ())),
        preferred_element_type=jnp.float32,
    )
    acc = acc + b_ref[...][None, :]
    pre_ref[...] = acc
    cmax_ref[...] = jnp.max(
        acc.reshape(TOK_BLK, DICT_BLK // CHUNK, CHUNK), axis=2)


def _encode(x_flat, W_enc, b_enc):
    nt = x_flat.shape[0]
    grid = (DS // DICT_BLK, nt // TOK_BLK)
    return pl.pallas_call(
        _encode_body,
        grid=grid,
        in_specs=[
            pl.BlockSpec((TOK_BLK, DM), lambda d, t: (t, 0)),
            pl.BlockSpec((DICT_BLK, DM), lambda d, t: (d, 0)),
            pl.BlockSpec((DICT_BLK,), lambda d, t: (d,)),
        ],
        out_specs=[
            pl.BlockSpec((TOK_BLK, DICT_BLK), lambda d, t: (t, d)),
            pl.BlockSpec((TOK_BLK, DICT_BLK // CHUNK), lambda d, t: (t, d)),
        ],
        out_shape=[
            jax.ShapeDtypeStruct((nt, DS), jnp.float32),
            jax.ShapeDtypeStruct((nt, DS // CHUNK), jnp.float32),
        ],
    )(x_flat, W_enc, b_enc)


DEC_TOK = 256


def _decode_body(f_ref, w_ref, b_ref, out_ref, acc_ref):
    k = pl.program_id(0)
    t = pl.program_id(1)
    nk = pl.num_programs(0)
    prod = jax.lax.dot_general(
        f_ref[...].astype(jnp.bfloat16), w_ref[...].astype(jnp.bfloat16),
        dimension_numbers=(((1,), (1,)), ((), ())),
        preferred_element_type=jnp.float32,
    )
    sl = pl.ds(t * DEC_TOK, DEC_TOK)

    @pl.when(k == 0)
    def _():
        acc_ref[sl, :] = prod

    @pl.when(k != 0)
    def _():
        acc_ref[sl, :] += prod

    @pl.when(k == nk - 1)
    def _():
        out_ref[...] = acc_ref[sl, :] + b_ref[...][None, :]


def _decode(features, W_dec, b_dec):
    nt = features.shape[0]
    grid = (DS // DEC_KBLK, nt // DEC_TOK)
    return pl.pallas_call(
        _decode_body,
        grid=grid,
        in_specs=[
            pl.BlockSpec((DEC_TOK, DEC_KBLK), lambda k, t: (t, k)),
            pl.BlockSpec((DM, DEC_KBLK), lambda k, t: (0, k)),
            pl.BlockSpec((DM,), lambda k, t: (0,)),
        ],
        out_specs=pl.BlockSpec((DEC_TOK, DM), lambda k, t: (t, 0)),
        out_shape=jax.ShapeDtypeStruct((nt, DM), jnp.float32),
        scratch_shapes=[pltpu.VMEM((2048, DM), jnp.float32)],
        compiler_params=pltpu.CompilerParams(
            dimension_semantics=("arbitrary", "arbitrary"),
        ),
    )(features, W_dec, b_dec)


NEG = -3.0e38
NW = 32
RPW = 2048 // NW
NCHV = (DS // CHUNK) // 16
POOL = 14


def _splat_lane(v, lane):
    idx = jnp.full((16,), lane, jnp.int32)
    return jnp.take(v, idx)


def _merge2(a, b, sort_loser):
    from jax.experimental.pallas import tpu_sc as plsc
    av, ai = a
    bv, bi = b
    rbv = lax.rev(bv, (0,))
    rbi = lax.rev(bi, (0,))
    c = av >= rbv
    hv = jnp.where(c, av, rbv)
    hi = jnp.where(c, ai, rbi)
    lv = jnp.where(c, rbv, av)
    li = jnp.where(c, rbi, ai)
    wv, wi = plsc.sort_key_val(hv, hi, descending=True)
    if sort_loser:
        lv, li = plsc.sort_key_val(lv, li, descending=True)
    return (wv, wi), (lv, li)


def _tree_top16(leaves, collect_losers):
    losers = []
    cur = list(leaves)
    while len(cur) > 1:
        nxt = []
        for i in range(0, len(cur) - 1, 2):
            w, l = _merge2(cur[i], cur[i + 1], sort_loser=collect_losers)
            nxt.append(w)
            if collect_losers:
                losers.append(l)
        if len(cur) % 2:
            nxt.append(cur[-1])
        cur = nxt
    return cur[0], losers


def _compact(pool_v, pool_i, Av, Ai, Bv, Bi):
    from jax.experimental.pallas import tpu_sc as plsc
    leaves = []
    for s in range(POOL):
        pv = pool_v[pl.ds(s * 16, 16)]
        pi = pool_i[pl.ds(s * 16, 16)]
        leaves.append((pv, pi))
    leaves.append((Av, Ai))
    leaves.append((Bv, Bi))
    (nAv, nAi), losers = _tree_top16(leaves, collect_losers=True)
    (nBv, nBi), _ = _tree_top16(losers, collect_losers=False)
    thr = _splat_lane(nBv, 15)
    return nAv, nAi, nBv, nBi, thr


def _scan_level(read_vreg, read_idx, nv, pool_v, pool_i, thr0, Av, Ai, Bv, Bi):
    slot0 = jnp.int32(0)

    @pl.loop(0, nv, init_carry=(slot0, thr0, Av, Ai, Bv, Bi))
    def scan(j, carry):
        slot, thr, av, ai, bv, bi = carry
        from jax.experimental.pallas import tpu_sc as plsc
        v = read_vreg(j)
        vs, is_ = plsc.sort_key_val(v, read_idx(j), descending=True)
        hit = vs[0] >= thr[0]

        def on_hit(slot, thr, av, ai, bv, bi):
            vm = jnp.where(vs >= thr, vs, NEG)
            pool_v[pl.ds(slot * 16, 16)] = vm
            pool_i[pl.ds(slot * 16, 16)] = is_
            slot = slot + 1

            def do_compact(av, ai, bv, bi):
                nav, nai, nbv, nbi, nthr = _compact(
                    pool_v, pool_i, av, ai, bv, bi)
                return jnp.int32(0), nthr, nav, nai, nbv, nbi

            def no_compact(av, ai, bv, bi):
                return slot, thr, av, ai, bv, bi

            return lax.cond(slot == POOL, do_compact, no_compact,
                            av, ai, bv, bi)

        def no_hit(slot, thr, av, ai, bv, bi):
            return slot, thr, av, ai, bv, bi

        return lax.cond(hit, on_hit, no_hit, slot, thr, av, ai, bv, bi)

    slot, thr, av, ai, bv, bi = scan
    neg16 = jnp.broadcast_to(NEG, (16,))
    zero16 = jnp.zeros((16,), jnp.int32)

    @pl.loop(slot, POOL)
    def _(s):
        pool_v[pl.ds(s * 16, 16)] = neg16
        pool_i[pl.ds(s * 16, 16)] = zero16

    def do_final(av, ai, bv, bi):
        nav, nai, nbv, nbi, nthr = _compact(pool_v, pool_i, av, ai, bv, bi)
        return nthr, nav, nai, nbv, nbi

    def no_final(av, ai, bv, bi):
        return thr, av, ai, bv, bi

    return lax.cond(slot > 0, do_final, no_final, av, ai, bv, bi)


def _sc_topk_body(cmax_hbm, prech_hbm, feat_hbm,
                  cm_buf, pool_v, pool_i, gidx, gbuf, eid, feat_buf,
                  last_ids, sem_g, sem_w0, sem_w1):
    from jax.experimental.pallas import tpu_sc as plsc
    wid = lax.axis_index("s") * 2 + lax.axis_index("c")
    iota = lax.iota(jnp.int32, 16)
    neg16 = jnp.broadcast_to(NEG, (16,))
    zero16i = jnp.zeros((16,), jnp.int32)
    zero16f = jnp.zeros((16,), jnp.float32)

    @pl.loop(0, 2 * DS // 16)
    def _(i):
        feat_buf[pl.ds(i * 16, 16)] = zero16f

    last_ids[pl.ds(0, 16)] = iota
    last_ids[pl.ds(16, 16)] = iota + 16
    last_ids[pl.ds(32, 16)] = iota
    last_ids[pl.ds(48, 16)] = iota + 16

    @pl.loop(0, RPW)
    def _(rr):
        row = wid * RPW + rr
        pltpu.sync_copy(cmax_hbm.at[row], cm_buf)

        thr, av, ai, bv, bi = _scan_level(
            lambda j: cm_buf[pl.ds(j * 16, 16)],
            lambda j: iota + j * 16,
            NCHV, pool_v, pool_i,
            neg16, neg16, zero16i, neg16, zero16i)

        base = row * (DS // CHUNK)
        gidx[pl.ds(0, 16)] = ai + base
        gidx[pl.ds(16, 16)] = bi + base
        pltpu.async_copy(prech_hbm.at[gidx], gbuf, sem_g).wait()

        for c in range(16):
            eid[pl.ds(c * 16, 16)] = _splat_lane(ai, c) * 16 + iota
            eid[pl.ds((c + 16) * 16, 16)] = _splat_lane(bi, c) * 16 + iota

        def read_v(c):
            return gbuf[c]

        def read_idx2(c):
            return eid[pl.ds(c * 16, 16)]

        thr2, av2, ai2, bv2, bi2 = _scan_level(
            read_v, read_idx2, 32, pool_v, pool_i,
            thr, neg16, zero16i, neg16, zero16i)

        b = rr % 2
        off = b * DS

        def wait_prev(_):
            prev_row = row - 2
            pltpu.make_async_copy(
                feat_buf.at[pl.ds(off, DS)], feat_hbm.at[prev_row],
                sem_w0).wait()
            return 0

        def wait_prev1(_):
            prev_row = row - 2
            pltpu.make_async_copy(
                feat_buf.at[pl.ds(off, DS)], feat_hbm.at[prev_row],
                sem_w1).wait()
            return 0

        def no_wait(_):
            return 0

        _ = lax.cond(rr >= 2,
                     lambda _: lax.cond(b == 0, wait_prev, wait_prev1, 0),
                     no_wait, 0)

        li0 = last_ids[pl.ds(b * 32, 16)]
        li1 = last_ids[pl.ds(b * 32 + 16, 16)]
        plsc.store_scatter(feat_buf, [li0 + off], zero16f)
        plsc.store_scatter(feat_buf, [li1 + off], zero16f)
        va = jnp.maximum(av2, 0.0)
        vb = jnp.maximum(bv2, 0.0)
        plsc.store_scatter(feat_buf, [ai2 + off], va)
        plsc.store_scatter(feat_buf, [bi2 + off], vb)
        last_ids[pl.ds(b * 32, 16)] = ai2
        last_ids[pl.ds(b * 32 + 16, 16)] = bi2

        def start0(_):
            pltpu.async_copy(feat_buf.at[pl.ds(off, DS)],
                             feat_hbm.at[row], sem_w0)
            return 0

        def start1(_):
            pltpu.async_copy(feat_buf.at[pl.ds(off, DS)],
                             feat_hbm.at[row], sem_w1)
            return 0

        _ = lax.cond(b == 0, start0, start1, 0)

    last0 = wid * RPW + RPW - 2
    pltpu.make_async_copy(feat_buf.at[pl.ds(0, DS)],
                          feat_hbm.at[last0], sem_w0).wait()
    pltpu.make_async_copy(feat_buf.at[pl.ds(DS, DS)],
                          feat_hbm.at[last0 + 1], sem_w1).wait()


def _sc_topk(cmax, pre_chunks):
    from jax.experimental.pallas import tpu_sc as plsc
    mesh = plsc.VectorSubcoreMesh(core_axis_name="c", subcore_axis_name="s",
                                  num_cores=2, num_subcores=16)
    nt = cmax.shape[0]
    return pl.kernel(
        _sc_topk_body,
        out_type=jax.ShapeDtypeStruct((nt, DS), jnp.float32),
        mesh=mesh,
        compiler_params=pltpu.CompilerParams(
            needs_layout_passes=False, use_tc_tiling_on_sc=False),
        scratch_types=[
            pltpu.VMEM((DS // CHUNK,), jnp.float32),
            pltpu.VMEM((POOL * 16,), jnp.float32),
            pltpu.VMEM((POOL * 16,), jnp.int32),
            pltpu.VMEM((32,), jnp.int32),
            pltpu.VMEM((32, 16), jnp.float32),
            pltpu.VMEM((32 * 16,), jnp.int32),
            pltpu.VMEM((2 * DS,), jnp.float32),
            pltpu.VMEM((64,), jnp.int32),
            pltpu.SemaphoreType.DMA,
            pltpu.SemaphoreType.DMA,
            pltpu.SemaphoreType.DMA,
        ],
    )(cmax, pre_chunks)


def _topk_features(pre_acts, cmax):
    pre_chunks = pre_acts.reshape(-1, CHUNK)
    return _sc_topk(cmax, pre_chunks)


def kernel(x, W_enc, b_enc, W_dec, b_dec):
    orig_shape = x.shape
    x_flat = x.reshape(-1, DM)
    pre_acts, cmax = _encode(x_flat, W_enc, b_enc)
    features = _topk_features(pre_acts, cmax)
    recon = _decode(features, W_dec, b_dec)
    return (
        recon.reshape(orig_shape),
        features.reshape(orig_shape[:-1] + (DS,)),
        pre_acts.reshape(orig_shape[:-1] + (DS,)),
    )


# speedup vs baseline: 3.3150x; 1.0075x over previous
import functools

import jax
import jax.numpy as jnp
from jax import lax
from jax.experimental import pallas as pl
from jax.experimental.pallas import tpu as pltpu

DM = 2048
DS = 32768
TK = 32
CHUNK = 16

TOK_BLK = 128
DICT_BLK = 2048
DEC_KBLK = 1024


def _encode_body(x_ref, w_ref, b_ref, pre_ref, cmax_ref):
    acc = jax.lax.dot_general(
        x_ref[...], w_ref[...],
        dimension_numbers=(((1,), (1,)), ((), ())),
        preferred_element_type=jnp.float32,
    )
    acc = acc + b_ref[...][None, :]
    pre_ref[...] = acc
    cmax_ref[...] = jnp.max(
        acc.reshape(TOK_BLK, DICT_BLK // CHUNK, CHUNK), axis=2)


def _encode(x_flat, W_enc, b_enc):
    nt = x_flat.shape[0]
    grid = (DS // DICT_BLK, nt // TOK_BLK)
    return pl.pallas_call(
        _encode_body,
        grid=grid,
        in_specs=[
            pl.BlockSpec((TOK_BLK, DM), lambda d, t: (t, 0)),
            pl.BlockSpec((DICT_BLK, DM), lambda d, t: (d, 0)),
            pl.BlockSpec((DICT_BLK,), lambda d, t: (d,)),
        ],
        out_specs=[
            pl.BlockSpec((TOK_BLK, DICT_BLK), lambda d, t: (t, d)),
            pl.BlockSpec((TOK_BLK, DICT_BLK // CHUNK), lambda d, t: (t, d)),
        ],
        out_shape=[
            jax.ShapeDtypeStruct((nt, DS), jnp.float32),
            jax.ShapeDtypeStruct((nt, DS // CHUNK), jnp.float32),
        ],
    )(x_flat, W_enc, b_enc)


DEC_TOK = 256


def _decode_body(f_ref, w_ref, b_ref, out_ref, acc_ref):
    k = pl.program_id(0)
    t = pl.program_id(1)
    nk = pl.num_programs(0)
    prod = jax.lax.dot_general(
        f_ref[...].astype(jnp.bfloat16), w_ref[...].astype(jnp.bfloat16),
        dimension_numbers=(((1,), (1,)), ((), ())),
        preferred_element_type=jnp.float32,
    )
    sl = pl.ds(t * DEC_TOK, DEC_TOK)

    @pl.when(k == 0)
    def _():
        acc_ref[sl, :] = prod

    @pl.when(k != 0)
    def _():
        acc_ref[sl, :] += prod

    @pl.when(k == nk - 1)
    def _():
        out_ref[...] = acc_ref[sl, :] + b_ref[...][None, :]


def _decode(features, W_dec, b_dec):
    nt = features.shape[0]
    grid = (DS // DEC_KBLK, nt // DEC_TOK)
    return pl.pallas_call(
        _decode_body,
        grid=grid,
        in_specs=[
            pl.BlockSpec((DEC_TOK, DEC_KBLK), lambda k, t: (t, k)),
            pl.BlockSpec((DM, DEC_KBLK), lambda k, t: (0, k)),
            pl.BlockSpec((DM,), lambda k, t: (0,)),
        ],
        out_specs=pl.BlockSpec((DEC_TOK, DM), lambda k, t: (t, 0)),
        out_shape=jax.ShapeDtypeStruct((nt, DM), jnp.float32),
        scratch_shapes=[pltpu.VMEM((nt, DM), jnp.float32)],
        compiler_params=pltpu.CompilerParams(
            dimension_semantics=("arbitrary", "arbitrary"),
        ),
    )(features, W_dec, b_dec)


NEG = -3.0e38
NW = 32
RPW = 2048 // NW
NCHV = (DS // CHUNK) // 16
POOL = 14


def _splat_lane(v, lane):
    idx = jnp.full((16,), lane, jnp.int32)
    return jnp.take(v, idx)


def _merge2(a, b, sort_loser):
    from jax.experimental.pallas import tpu_sc as plsc
    av, ai = a
    bv, bi = b
    rbv = lax.rev(bv, (0,))
    rbi = lax.rev(bi, (0,))
    c = av >= rbv
    hv = jnp.where(c, av, rbv)
    hi = jnp.where(c, ai, rbi)
    lv = jnp.where(c, rbv, av)
    li = jnp.where(c, rbi, ai)
    wv, wi = plsc.sort_key_val(hv, hi, descending=True)
    if sort_loser:
        lv, li = plsc.sort_key_val(lv, li, descending=True)
    return (wv, wi), (lv, li)


def _tree_top16(leaves, collect_losers):
    losers = []
    cur = list(leaves)
    while len(cur) > 1:
        nxt = []
        for i in range(0, len(cur) - 1, 2):
            w, l = _merge2(cur[i], cur[i + 1], sort_loser=collect_losers)
            nxt.append(w)
            if collect_losers:
                losers.append(l)
        if len(cur) % 2:
            nxt.append(cur[-1])
        cur = nxt
    return cur[0], losers


def _compact(pool_v, pool_i, Av, Ai, Bv, Bi):
    from jax.experimental.pallas import tpu_sc as plsc
    leaves = []
    for s in range(POOL):
        pv = pool_v[pl.ds(s * 16, 16)]
        pi = pool_i[pl.ds(s * 16, 16)]
        leaves.append((pv, pi))
    leaves.append((Av, Ai))
    leaves.append((Bv, Bi))
    (nAv, nAi), losers = _tree_top16(leaves, collect_losers=True)
    (nBv, nBi), _ = _tree_top16(losers, collect_losers=False)
    thr = _splat_lane(nBv, 15)
    return nAv, nAi, nBv, nBi, thr


def _scan_level(read_vreg, read_idx, nv, pool_v, pool_i, thr0, Av, Ai, Bv, Bi):
    slot0 = jnp.int32(0)

    @pl.loop(0, nv, init_carry=(slot0, thr0, Av, Ai, Bv, Bi))
    def scan(j, carry):
        slot, thr, av, ai, bv, bi = carry
        from jax.experimental.pallas import tpu_sc as plsc
        v = read_vreg(j)
        vs, is_ = plsc.sort_key_val(v, read_idx(j), descending=True)
        hit = vs[0] >= thr[0]

        def on_hit(slot, thr, av, ai, bv, bi):
            vm = jnp.where(vs >= thr, vs, NEG)
            pool_v[pl.ds(slot * 16, 16)] = vm
            pool_i[pl.ds(slot * 16, 16)] = is_
            slot = slot + 1

            def do_compact(av, ai, bv, bi):
                nav, nai, nbv, nbi, nthr = _compact(
                    pool_v, pool_i, av, ai, bv, bi)
                return jnp.int32(0), nthr, nav, nai, nbv, nbi

            def no_compact(av, ai, bv, bi):
                return slot, thr, av, ai, bv, bi

            return lax.cond(slot == POOL, do_compact, no_compact,
                            av, ai, bv, bi)

        def no_hit(slot, thr, av, ai, bv, bi):
            return slot, thr, av, ai, bv, bi

        return lax.cond(hit, on_hit, no_hit, slot, thr, av, ai, bv, bi)

    slot, thr, av, ai, bv, bi = scan
    neg16 = jnp.broadcast_to(NEG, (16,))
    zero16 = jnp.zeros((16,), jnp.int32)

    @pl.loop(slot, POOL)
    def _(s):
        pool_v[pl.ds(s * 16, 16)] = neg16
        pool_i[pl.ds(s * 16, 16)] = zero16

    def do_final(av, ai, bv, bi):
        nav, nai, nbv, nbi, nthr = _compact(pool_v, pool_i, av, ai, bv, bi)
        return nthr, nav, nai, nbv, nbi

    def no_final(av, ai, bv, bi):
        return thr, av, ai, bv, bi

    return lax.cond(slot > 0, do_final, no_final, av, ai, bv, bi)


def _sc_topk_body(RPW, cmax_hbm, prech_hbm, feat_hbm,
                  cm_buf, pool_v, pool_i, gidx, gbuf, eid, feat_buf,
                  last_ids, sem_g, sem_w0, sem_w1):
    from jax.experimental.pallas import tpu_sc as plsc
    wid = lax.axis_index("s") * 2 + lax.axis_index("c")
    iota = lax.iota(jnp.int32, 16)
    neg16 = jnp.broadcast_to(NEG, (16,))
    zero16i = jnp.zeros((16,), jnp.int32)
    zero16f = jnp.zeros((16,), jnp.float32)

    @pl.loop(0, 2 * DS // 16)
    def _(i):
        feat_buf[pl.ds(i * 16, 16)] = zero16f

    last_ids[pl.ds(0, 16)] = iota
    last_ids[pl.ds(16, 16)] = iota + 16
    last_ids[pl.ds(32, 16)] = iota
    last_ids[pl.ds(48, 16)] = iota + 16

    @pl.loop(0, RPW)
    def _(rr):
        row = wid * RPW + rr
        pltpu.sync_copy(cmax_hbm.at[row], cm_buf)

        thr, av, ai, bv, bi = _scan_level(
            lambda j: cm_buf[pl.ds(j * 16, 16)],
            lambda j: iota + j * 16,
            NCHV, pool_v, pool_i,
            neg16, neg16, zero16i, neg16, zero16i)

        base = row * (DS // CHUNK)
        gidx[pl.ds(0, 16)] = ai + base
        gidx[pl.ds(16, 16)] = bi + base
        pltpu.async_copy(prech_hbm.at[gidx], gbuf, sem_g).wait()

        for c in range(16):
            eid[pl.ds(c * 16, 16)] = _splat_lane(ai, c) * 16 + iota
            eid[pl.ds((c + 16) * 16, 16)] = _splat_lane(bi, c) * 16 + iota

        def read_v(c):
            return gbuf[c]

        def read_idx2(c):
            return eid[pl.ds(c * 16, 16)]

        thr2, av2, ai2, bv2, bi2 = _scan_level(
            read_v, read_idx2, 32, pool_v, pool_i,
            thr, neg16, zero16i, neg16, zero16i)

        b = rr % 2
        off = b * DS

        def wait_prev(_):
            prev_row = row - 2
            pltpu.make_async_copy(
                feat_buf.at[pl.ds(off, DS)], feat_hbm.at[prev_row],
                sem_w0).wait()
            return 0

        def wait_prev1(_):
            prev_row = row - 2
            pltpu.make_async_copy(
                feat_buf.at[pl.ds(off, DS)], feat_hbm.at[prev_row],
                sem_w1).wait()
            return 0

        def no_wait(_):
            return 0

        _ = lax.cond(rr >= 2,
                     lambda _: lax.cond(b == 0, wait_prev, wait_prev1, 0),
                     no_wait, 0)

        li0 = last_ids[pl.ds(b * 32, 16)]
        li1 = last_ids[pl.ds(b * 32 + 16, 16)]
        plsc.store_scatter(feat_buf, [li0 + off], zero16f)
        plsc.store_scatter(feat_buf, [li1 + off], zero16f)
        va = jnp.maximum(av2, 0.0)
        vb = jnp.maximum(bv2, 0.0)
        plsc.store_scatter(feat_buf, [ai2 + off], va)
        plsc.store_scatter(feat_buf, [bi2 + off], vb)
        last_ids[pl.ds(b * 32, 16)] = ai2
        last_ids[pl.ds(b * 32 + 16, 16)] = bi2

        def start0(_):
            pltpu.async_copy(feat_buf.at[pl.ds(off, DS)],
                             feat_hbm.at[row], sem_w0)
            return 0

        def start1(_):
            pltpu.async_copy(feat_buf.at[pl.ds(off, DS)],
                             feat_hbm.at[row], sem_w1)
            return 0

        _ = lax.cond(b == 0, start0, start1, 0)

    last0 = wid * RPW + RPW - 2
    pltpu.make_async_copy(feat_buf.at[pl.ds(0, DS)],
                          feat_hbm.at[last0], sem_w0).wait()
    pltpu.make_async_copy(feat_buf.at[pl.ds(DS, DS)],
                          feat_hbm.at[last0 + 1], sem_w1).wait()


def _sc_topk(cmax, pre_chunks):
    from jax.experimental.pallas import tpu_sc as plsc
    mesh = plsc.VectorSubcoreMesh(core_axis_name="c", subcore_axis_name="s",
                                  num_cores=2, num_subcores=16)
    nt = cmax.shape[0]
    return pl.kernel(
        functools.partial(_sc_topk_body, nt // NW),
        out_type=jax.ShapeDtypeStruct((nt, DS), jnp.float32),
        mesh=mesh,
        compiler_params=pltpu.CompilerParams(
            needs_layout_passes=False, use_tc_tiling_on_sc=False),
        scratch_types=[
            pltpu.VMEM((DS // CHUNK,), jnp.float32),
            pltpu.VMEM((POOL * 16,), jnp.float32),
            pltpu.VMEM((POOL * 16,), jnp.int32),
            pltpu.VMEM((32,), jnp.int32),
            pltpu.VMEM((32, 16), jnp.float32),
            pltpu.VMEM((32 * 16,), jnp.int32),
            pltpu.VMEM((2 * DS,), jnp.float32),
            pltpu.VMEM((64,), jnp.int32),
            pltpu.SemaphoreType.DMA,
            pltpu.SemaphoreType.DMA,
            pltpu.SemaphoreType.DMA,
        ],
    )(cmax, pre_chunks)


def _topk_features(pre_acts, cmax):
    pre_chunks = pre_acts.reshape(-1, CHUNK)
    return _sc_topk(cmax, pre_chunks)


NCHUNKS = 2


def kernel(x, W_enc, b_enc, W_dec, b_dec):
    orig_shape = x.shape
    x_flat = x.reshape(-1, DM)
    nt = x_flat.shape[0]
    pre_acts, cmax = _encode(x_flat, W_enc, b_enc)
    csz = nt // NCHUNKS
    feats, recons = [], []
    for c in range(NCHUNKS):
        sl = slice(c * csz, (c + 1) * csz)
        f = _sc_topk(cmax[sl], pre_acts[sl].reshape(-1, CHUNK))
        feats.append(f)
        recons.append(_decode(f, W_dec, b_dec))
    features = jnp.concatenate(feats) if NCHUNKS > 1 else feats[0]
    recon = jnp.concatenate(recons) if NCHUNKS > 1 else recons[0]
    return (
        recon.reshape(orig_shape),
        features.reshape(orig_shape[:-1] + (DS,)),
        pre_acts.reshape(orig_shape[:-1] + (DS,)),
    )


# speedup vs baseline: 3.5174x; 1.0611x over previous
import functools

import jax
import jax.numpy as jnp
from jax import lax
from jax.experimental import pallas as pl
from jax.experimental.pallas import tpu as pltpu

DM = 2048
DS = 32768
TK = 32
CHUNK = 16

TOK_BLK = 128
DICT_BLK = 2048
DEC_KBLK = 1024


def _encode_body(x_ref, w_ref, b_ref, pre_ref, cmax_ref):
    acc = jax.lax.dot_general(
        x_ref[...], w_ref[...],
        dimension_numbers=(((1,), (1,)), ((), ())),
        preferred_element_type=jnp.float32,
    )
    acc = acc + b_ref[...][None, :]
    pre_ref[...] = acc
    cmax_ref[...] = jnp.max(
        acc.reshape(TOK_BLK, DICT_BLK // CHUNK, CHUNK), axis=2)


def _encode(x_flat, W_enc, b_enc):
    nt = x_flat.shape[0]
    grid = (DS // DICT_BLK, nt // TOK_BLK)
    return pl.pallas_call(
        _encode_body,
        grid=grid,
        in_specs=[
            pl.BlockSpec((TOK_BLK, DM), lambda d, t: (t, 0)),
            pl.BlockSpec((DICT_BLK, DM), lambda d, t: (d, 0)),
            pl.BlockSpec((DICT_BLK,), lambda d, t: (d,)),
        ],
        out_specs=[
            pl.BlockSpec((TOK_BLK, DICT_BLK), lambda d, t: (t, d)),
            pl.BlockSpec((TOK_BLK, DICT_BLK // CHUNK), lambda d, t: (t, d)),
        ],
        out_shape=[
            jax.ShapeDtypeStruct((nt, DS), jnp.float32),
            jax.ShapeDtypeStruct((nt, DS // CHUNK), jnp.float32),
        ],
    )(x_flat, W_enc, b_enc)


DEC_TOK = 256


def _decode_body(f_ref, w_ref, b_ref, out_ref, acc_ref):
    k = pl.program_id(0)
    t = pl.program_id(1)
    nk = pl.num_programs(0)
    prod = jax.lax.dot_general(
        f_ref[...].astype(jnp.bfloat16), w_ref[...].astype(jnp.bfloat16),
        dimension_numbers=(((1,), (1,)), ((), ())),
        preferred_element_type=jnp.float32,
    )
    sl = pl.ds(t * DEC_TOK, DEC_TOK)

    @pl.when(k == 0)
    def _():
        acc_ref[sl, :] = prod

    @pl.when(k != 0)
    def _():
        acc_ref[sl, :] += prod

    @pl.when(k == nk - 1)
    def _():
        out_ref[...] = acc_ref[sl, :] + b_ref[...][None, :]


def _decode(features, W_dec, b_dec):
    nt = features.shape[0]
    grid = (DS // DEC_KBLK, nt // DEC_TOK)
    return pl.pallas_call(
        _decode_body,
        grid=grid,
        in_specs=[
            pl.BlockSpec((DEC_TOK, DEC_KBLK), lambda k, t: (t, k)),
            pl.BlockSpec((DM, DEC_KBLK), lambda k, t: (0, k)),
            pl.BlockSpec((DM,), lambda k, t: (0,)),
        ],
        out_specs=pl.BlockSpec((DEC_TOK, DM), lambda k, t: (t, 0)),
        out_shape=jax.ShapeDtypeStruct((nt, DM), jnp.float32),
        scratch_shapes=[pltpu.VMEM((nt, DM), jnp.float32)],
        compiler_params=pltpu.CompilerParams(
            dimension_semantics=("arbitrary", "arbitrary"),
        ),
    )(features, W_dec, b_dec)


NEG = -3.0e38
NW = 32
RPW = 2048 // NW
NCHV = (DS // CHUNK) // 16
POOL = 16


def _splat_lane(v, lane):
    idx = jnp.full((16,), lane, jnp.int32)
    return jnp.take(v, idx)


def _merge2(a, b, sort_loser):
    from jax.experimental.pallas import tpu_sc as plsc
    av, ai = a
    bv, bi = b
    rbv = lax.rev(bv, (0,))
    rbi = lax.rev(bi, (0,))
    c = av >= rbv
    hv = jnp.where(c, av, rbv)
    hi = jnp.where(c, ai, rbi)
    lv = jnp.where(c, rbv, av)
    li = jnp.where(c, rbi, ai)
    wv, wi = plsc.sort_key_val(hv, hi, descending=True)
    if sort_loser:
        lv, li = plsc.sort_key_val(lv, li, descending=True)
    return (wv, wi), (lv, li)


def _tree_top16(leaves, collect_losers):
    losers = []
    cur = list(leaves)
    while len(cur) > 1:
        nxt = []
        for i in range(0, len(cur) - 1, 2):
            w, l = _merge2(cur[i], cur[i + 1], sort_loser=collect_losers)
            nxt.append(w)
            if collect_losers:
                losers.append(l)
        if len(cur) % 2:
            nxt.append(cur[-1])
        cur = nxt
    return cur[0], losers


def _compact(pool_v, pool_i, Av, Ai, Bv, Bi):
    from jax.experimental.pallas import tpu_sc as plsc
    leaves = []
    for s in range(POOL):
        pv = pool_v[pl.ds(s * 16, 16)]
        pi = pool_i[pl.ds(s * 16, 16)]
        leaves.append((pv, pi))
    leaves.append((Av, Ai))
    leaves.append((Bv, Bi))
    (nAv, nAi), losers = _tree_top16(leaves, collect_losers=True)
    (nBv, nBi), _ = _tree_top16(losers, collect_losers=False)
    thr = _splat_lane(nBv, 15)
    return nAv, nAi, nBv, nBi, thr


def _scan_level(read_vreg, read_idx, nv, pool_v, pool_i, thr0, Av, Ai, Bv, Bi):
    slot0 = jnp.int32(0)

    B = 8
    assert nv % B == 0

    @pl.loop(0, nv // B, init_carry=(slot0, thr0, Av, Ai, Bv, Bi))
    def scan(jb, carry):
        slot, thr, av, ai, bv, bi = carry
        from jax.experimental.pallas import tpu_sc as plsc
        vs = [read_vreg(jb * B + k) for k in range(B)]
        mx = vs[0]
        for k in range(1, B):
            mx = jnp.maximum(mx, vs[k])
        sm, _ = plsc.sort_key_val(mx, jnp.zeros((16,), jnp.int32),
                                  descending=True)
        hit = sm[0] >= thr[0]

        def on_hit(slot):
            s = slot
            for k in range(B):
                vks, iks = plsc.sort_key_val(
                    vs[k], read_idx(jb * B + k), descending=True)

                def store(s, vks=vks, iks=iks):
                    pool_v[pl.ds(s * 16, 16)] = jnp.where(
                        vks >= thr, vks, NEG)
                    pool_i[pl.ds(s * 16, 16)] = iks
                    return s + 1

                s = lax.cond(vks[0] >= thr[0], store, lambda s: s, s)
            return s

        slot = lax.cond(hit, on_hit, lambda s: s, slot)

        def do_compact(av, ai, bv, bi):
            neg16_ = jnp.broadcast_to(NEG, (16,))
            zero16_ = jnp.zeros((16,), jnp.int32)

            @pl.loop(slot, POOL)
            def _(s):
                pool_v[pl.ds(s * 16, 16)] = neg16_
                pool_i[pl.ds(s * 16, 16)] = zero16_

            nav, nai, nbv, nbi, nthr = _compact(
                pool_v, pool_i, av, ai, bv, bi)
            return jnp.int32(0), nthr, nav, nai, nbv, nbi

        def no_compact(av, ai, bv, bi):
            return slot, thr, av, ai, bv, bi

        return lax.cond(slot >= POOL - B, do_compact, no_compact,
                        av, ai, bv, bi)

    slot, thr, av, ai, bv, bi = scan
    neg16 = jnp.broadcast_to(NEG, (16,))
    zero16 = jnp.zeros((16,), jnp.int32)

    @pl.loop(slot, POOL)
    def _(s):
        pool_v[pl.ds(s * 16, 16)] = neg16
        pool_i[pl.ds(s * 16, 16)] = zero16

    def do_final(av, ai, bv, bi):
        nav, nai, nbv, nbi, nthr = _compact(pool_v, pool_i, av, ai, bv, bi)
        return nthr, nav, nai, nbv, nbi

    def no_final(av, ai, bv, bi):
        return thr, av, ai, bv, bi

    return lax.cond(slot > 0, do_final, no_final, av, ai, bv, bi)


def _sc_topk_body(RPW, cmax_hbm, prech_hbm, feat_hbm,
                  cm_buf, pool_v, pool_i, gidx, gbuf, eid, feat_buf,
                  last_ids, sem_g, sem_w0, sem_w1, sem_c0, sem_c1):
    from jax.experimental.pallas import tpu_sc as plsc
    wid = lax.axis_index("s") * 2 + lax.axis_index("c")
    NCH = DS // CHUNK
    iota = lax.iota(jnp.int32, 16)
    neg16 = jnp.broadcast_to(NEG, (16,))
    zero16i = jnp.zeros((16,), jnp.int32)
    zero16f = jnp.zeros((16,), jnp.float32)

    @pl.loop(0, 2 * DS // 16)
    def _(i):
        feat_buf[pl.ds(i * 16, 16)] = zero16f

    last_ids[pl.ds(0, 16)] = iota
    last_ids[pl.ds(16, 16)] = iota + 16
    last_ids[pl.ds(32, 16)] = iota
    last_ids[pl.ds(48, 16)] = iota + 16

    pltpu.async_copy(cmax_hbm.at[wid * RPW], cm_buf.at[pl.ds(0, NCH)],
                     sem_c0)

    @pl.loop(0, RPW)
    def _(rr):
        row = wid * RPW + rr
        cb = rr % 2
        cmoff = cb * NCH

        def pref0(_):
            pltpu.async_copy(cmax_hbm.at[row + 1],
                             cm_buf.at[pl.ds((1 - cb) * NCH, NCH)], sem_c1)
            return 0

        def pref1(_):
            pltpu.async_copy(cmax_hbm.at[row + 1],
                             cm_buf.at[pl.ds((1 - cb) * NCH, NCH)], sem_c0)
            return 0

        _ = lax.cond(rr + 1 < RPW,
                     lambda _: lax.cond(cb == 0, pref0, pref1, 0),
                     lambda _: 0, 0)

        def wait0(_):
            pltpu.make_async_copy(cmax_hbm.at[row],
                                  cm_buf.at[pl.ds(cmoff, NCH)],
                                  sem_c0).wait()
            return 0

        def wait1(_):
            pltpu.make_async_copy(cmax_hbm.at[row],
                                  cm_buf.at[pl.ds(cmoff, NCH)],
                                  sem_c1).wait()
            return 0

        _ = lax.cond(cb == 0, wait0, wait1, 0)

        thr, av, ai, bv, bi = _scan_level(
            lambda j: cm_buf[pl.ds(cmoff + j * 16, 16)],
            lambda j: iota + j * 16,
            NCHV, pool_v, pool_i,
            neg16, neg16, zero16i, neg16, zero16i)

        base = row * (DS // CHUNK)
        gidx[pl.ds(0, 16)] = ai + base
        gidx[pl.ds(16, 16)] = bi + base
        pltpu.async_copy(prech_hbm.at[gidx], gbuf, sem_g).wait()

        for c in range(16):
            eid[pl.ds(c * 16, 16)] = _splat_lane(ai, c) * 16 + iota
            eid[pl.ds((c + 16) * 16, 16)] = _splat_lane(bi, c) * 16 + iota

        def read_v(c):
            return gbuf[c]

        def read_idx2(c):
            return eid[pl.ds(c * 16, 16)]

        thr2, av2, ai2, bv2, bi2 = _scan_level(
            read_v, read_idx2, 32, pool_v, pool_i,
            thr, neg16, zero16i, neg16, zero16i)

        b = rr % 2
        off = b * DS

        def wait_prev(_):
            prev_row = row - 2
            pltpu.make_async_copy(
                feat_buf.at[pl.ds(off, DS)], feat_hbm.at[prev_row],
                sem_w0).wait()
            return 0

        def wait_prev1(_):
            prev_row = row - 2
            pltpu.make_async_copy(
                feat_buf.at[pl.ds(off, DS)], feat_hbm.at[prev_row],
                sem_w1).wait()
            return 0

        def no_wait(_):
            return 0

        _ = lax.cond(rr >= 2,
                     lambda _: lax.cond(b == 0, wait_prev, wait_prev1, 0),
                     no_wait, 0)

        li0 = last_ids[pl.ds(b * 32, 16)]
        li1 = last_ids[pl.ds(b * 32 + 16, 16)]
        plsc.store_scatter(feat_buf, [li0 + off], zero16f)
        plsc.store_scatter(feat_buf, [li1 + off], zero16f)
        va = jnp.maximum(av2, 0.0)
        vb = jnp.maximum(bv2, 0.0)
        plsc.store_scatter(feat_buf, [ai2 + off], va)
        plsc.store_scatter(feat_buf, [bi2 + off], vb)
        last_ids[pl.ds(b * 32, 16)] = ai2
        last_ids[pl.ds(b * 32 + 16, 16)] = bi2

        def start0(_):
            pltpu.async_copy(feat_buf.at[pl.ds(off, DS)],
                             feat_hbm.at[row], sem_w0)
            return 0

        def start1(_):
            pltpu.async_copy(feat_buf.at[pl.ds(off, DS)],
                             feat_hbm.at[row], sem_w1)
            return 0

        _ = lax.cond(b == 0, start0, start1, 0)

    last0 = wid * RPW + RPW - 2
    pltpu.make_async_copy(feat_buf.at[pl.ds(0, DS)],
                          feat_hbm.at[last0], sem_w0).wait()
    pltpu.make_async_copy(feat_buf.at[pl.ds(DS, DS)],
                          feat_hbm.at[last0 + 1], sem_w1).wait()


def _sc_topk(cmax, pre_chunks):
    from jax.experimental.pallas import tpu_sc as plsc
    mesh = plsc.VectorSubcoreMesh(core_axis_name="c", subcore_axis_name="s",
                                  num_cores=2, num_subcores=16)
    nt = cmax.shape[0]
    return pl.kernel(
        functools.partial(_sc_topk_body, nt // NW),
        out_type=jax.ShapeDtypeStruct((nt, DS), jnp.float32),
        mesh=mesh,
        compiler_params=pltpu.CompilerParams(
            needs_layout_passes=False, use_tc_tiling_on_sc=False),
        scratch_types=[
            pltpu.VMEM((2 * (DS // CHUNK),), jnp.float32),
            pltpu.VMEM((POOL * 16,), jnp.float32),
            pltpu.VMEM((POOL * 16,), jnp.int32),
            pltpu.VMEM((32,), jnp.int32),
            pltpu.VMEM((32, 16), jnp.float32),
            pltpu.VMEM((32 * 16,), jnp.int32),
            pltpu.VMEM((2 * DS,), jnp.float32),
            pltpu.VMEM((64,), jnp.int32),
            pltpu.SemaphoreType.DMA,
            pltpu.SemaphoreType.DMA,
            pltpu.SemaphoreType.DMA,
            pltpu.SemaphoreType.DMA,
            pltpu.SemaphoreType.DMA,
        ],
    )(cmax, pre_chunks)


def _topk_features(pre_acts, cmax):
    pre_chunks = pre_acts.reshape(-1, CHUNK)
    return _sc_topk(cmax, pre_chunks)


NCHUNKS = 2


def kernel(x, W_enc, b_enc, W_dec, b_dec):
    orig_shape = x.shape
    x_flat = x.reshape(-1, DM)
    nt = x_flat.shape[0]
    pre_acts, cmax = _encode(x_flat, W_enc, b_enc)
    csz = nt // NCHUNKS
    feats, recons = [], []
    for c in range(NCHUNKS):
        sl = slice(c * csz, (c + 1) * csz)
        f = _sc_topk(cmax[sl], pre_acts[sl].reshape(-1, CHUNK))
        feats.append(f)
        recons.append(_decode(f, W_dec, b_dec))
    features = jnp.concatenate(feats) if NCHUNKS > 1 else feats[0]
    recon = jnp.concatenate(recons) if NCHUNKS > 1 else recons[0]
    return (
        recon.reshape(orig_shape),
        features.reshape(orig_shape[:-1] + (DS,)),
        pre_acts.reshape(orig_shape[:-1] + (DS,)),
    )
